# baseline (speedup 1.0000x reference)
import functools

import jax
import jax.numpy as jnp
from jax import lax
from jax.experimental import pallas as pl
from jax.experimental.pallas import tpu as pltpu
from jax.experimental.pallas import tpu_sc as plsc


@functools.lru_cache(maxsize=None)
def _gather_fn(B, D, NC, NS, CH, NB, K):
    NW = NC * NS
    b_per_w = B // NW
    n_ch = b_per_w // CH
    n_grp = n_ch // NB
    mesh = plsc.VectorSubcoreMesh(core_axis_name="c", subcore_axis_name="s")

    @functools.partial(
        pl.kernel,
        mesh=mesh,
        out_type=jax.ShapeDtypeStruct((B, D), jnp.float32),
        scratch_types=[
            pltpu.VMEM((b_per_w,), jnp.int32),
            pltpu.VMEM((NB, CH, D), jnp.float32),
        ]
        + [pltpu.SemaphoreType.DMA] * (2 * NB),
        compiler_params=pltpu.CompilerParams(use_tc_tiling_on_sc=True),
    )
    def k(table_hbm, idx_hbm, out_hbm, idx_v, rows_v, *sems):
        gsems, osems = sems[:NB], sems[NB:]
        wid = lax.axis_index("s") * NC + lax.axis_index("c")
        base = wid * b_per_w
        pltpu.sync_copy(idx_hbm.at[pl.ds(base, b_per_w)], idx_v)

        def start_gather(j, b):
            pltpu.async_copy(
                table_hbm.at[idx_v.at[pl.ds(j * CH, CH)]], rows_v.at[b], gsems[b]
            )

        def wait_gather(j, b):
            pltpu.make_async_copy(
                table_hbm.at[idx_v.at[pl.ds(j * CH, CH)]], rows_v.at[b], gsems[b]
            ).wait()

        def wait_write(b):
            pltpu.make_async_copy(
                rows_v.at[b], out_hbm.at[pl.ds(base, CH)], osems[b]
            ).wait()

        for b in range(K):
            start_gather(b, b)

        def body(jo, carry):
            for b in range(NB):
                j = jo * NB + b
                bp = (b + K) % NB

                @pl.when((j + K < n_ch) & (j + K >= NB))
                def _():
                    wait_write(bp)

                @pl.when(j + K < n_ch)
                def _():
                    start_gather(j + K, bp)

                wait_gather(j, b)
                pltpu.async_copy(
                    rows_v.at[b], out_hbm.at[pl.ds(base + j * CH, CH)], osems[b]
                )

            return carry

        lax.fori_loop(0, n_grp, body, 0)

        for b in range(NB):
            wait_write(b)

    return k


def kernel(idx, table):
    B0, S = idx.shape
    V, D = table.shape
    B = B0 * S
    info = plsc.get_sparse_core_info()
    NC, NS = info.num_cores, info.num_subcores
    CH, NB, K = 64, 10, 7
    idx_t = idx.astype(jnp.int32).T.reshape(B)
    out = _gather_fn(B, D, NC, NS, CH, NB, K)(table, idx_t)
    return out.reshape(S, B0, D).transpose(1, 0, 2)
